# baseline (speedup 1.0000x reference)
import functools

import jax
import jax.numpy as jnp
from jax import lax
from jax.experimental import pallas as pl
from jax.experimental.pallas import tpu as pltpu
from jax.experimental.pallas import tpu_sc as plsc

BLANK = 0
NL = 16


def _argmax_body(len_ref, logits_hbm, out_ref, vbuf, sem, *, tblk, nt, nsteps):
    i = pl.program_id(0)
    b = i // nt
    t = i % nt
    V = logits_hbm.shape[-1]

    def needed(j):
        return (j % nt) * tblk < len_ref[j // nt]

    NBUF = 4
    LOOKAHEAD = NBUF - 1

    def start(j):
        return pltpu.make_async_copy(
            logits_hbm.at[j // nt, pl.ds((j % nt) * tblk, tblk)],
            vbuf.at[j % NBUF],
            sem.at[j % NBUF],
        )

    @pl.when(i == 0)
    def _():
        for j in range(min(LOOKAHEAD, nsteps)):
            @pl.when(needed(j))
            def _():
                start(j).start()

    nxt = jnp.minimum(i + LOOKAHEAD, nsteps - 1)
    @pl.when((i + LOOKAHEAD < nsteps) & needed(nxt))
    def _():
        start(nxt).start()

    @pl.when(needed(i))
    def _():
        start(i).wait()
        x = vbuf[i % NBUF]
        m = jnp.max(x, axis=-1, keepdims=True)
        iota_v = lax.broadcasted_iota(jnp.int32, (tblk, V), 1)
        am = jnp.min(jnp.where(x == m, iota_v, V), axis=-1, keepdims=True)
        tidx = t * tblk + lax.broadcasted_iota(jnp.int32, (tblk, 1), 0)
        out_ref[0, 0] = jnp.where(tidx < len_ref[b], am.astype(jnp.int32), BLANK)

    @pl.when(jnp.logical_not(needed(i)))
    def _():
        out_ref[0, 0] = jnp.zeros((tblk, 1), jnp.int32)


def _argmax_preds(logits, logits_len, tblk=512):
    B, T, V = logits.shape
    nt = T // tblk
    nsteps = B * nt
    out = pl.pallas_call(
        functools.partial(_argmax_body, tblk=tblk, nt=nt, nsteps=nsteps),
        grid=(nsteps,),
        in_specs=[
            pl.BlockSpec(memory_space=pltpu.SMEM),
            pl.BlockSpec(memory_space=pl.ANY),
        ],
        out_specs=pl.BlockSpec((1, 1, tblk, 1), lambda i: (i // nt, i % nt, 0, 0)),
        out_shape=jax.ShapeDtypeStruct((B, nt, tblk, 1), jnp.int32),
        scratch_shapes=[
            pltpu.VMEM((4, tblk, V), jnp.float32),
            pltpu.SemaphoreType.DMA((4,)),
        ],
    )(logits_len, logits)
    return out.reshape(B, T)


def _compact_body(preds_hbm, out_hbm, len_hbm, buf, orow, cbuf, *, B, T):
    c = lax.axis_index("c")
    s = lax.axis_index("s")
    wid = s * 2 + c

    @pl.when(wid < B)
    def _():
        b = wid
        buf[pl.ds(0, NL)] = jnp.zeros((NL,), jnp.int32)
        pltpu.sync_copy(preds_hbm.at[b], buf.at[pl.ds(NL, T)])
        neg1 = jnp.full((NL,), -1, jnp.int32)

        def body(i, cnt):
            base = NL + i * NL
            v = buf[pl.ds(base, NL)]
            prev = buf[pl.ds(base - 1, NL)]
            keep = (v != prev) & (v != BLANK)
            inc = plsc.cumsum(keep.astype(jnp.int32))
            posn = cnt + inc - 1
            orow[pl.ds(i * NL, NL)] = neg1
            plsc.store_scatter(orow, [posn], v, mask=keep)
            return cnt + plsc.all_reduce_population_count(keep)

        cnt = lax.fori_loop(0, T // NL, body, jnp.zeros((NL,), jnp.int32))
        cbuf[...] = cnt
        pltpu.sync_copy(orow, out_hbm.at[b])
        pltpu.sync_copy(cbuf, len_hbm.at[b])


def _compact(preds):
    B, T = preds.shape
    mesh = plsc.VectorSubcoreMesh(
        core_axis_name="c", subcore_axis_name="s", num_cores=2, num_subcores=16
    )
    f = pl.kernel(
        functools.partial(_compact_body, B=B, T=T),
        out_type=(
            jax.ShapeDtypeStruct((B, T), jnp.int32),
            jax.ShapeDtypeStruct((B, NL), jnp.int32),
        ),
        mesh=mesh,
        scratch_types=[
            pltpu.VMEM((NL + T,), jnp.int32),
            pltpu.VMEM((T,), jnp.int32),
            pltpu.VMEM((NL,), jnp.int32),
        ],
        compiler_params=pltpu.CompilerParams(use_tc_tiling_on_sc=False, needs_layout_passes=False),
    )
    return f(preds)


def kernel(logits, logits_len):
    preds = _argmax_preds(logits, logits_len)
    out, len2d = _compact(preds)
    return out, len2d[:, 0]


# speedup vs baseline: 1.1262x; 1.1262x over previous
import functools

import jax
import jax.numpy as jnp
from jax import lax
from jax.experimental import pallas as pl
from jax.experimental.pallas import tpu as pltpu
from jax.experimental.pallas import tpu_sc as plsc

BLANK = 0
NL = 16


def _argmax_body(len_ref, logits_hbm, out_ref, vbuf, sem, *, tblk, nt, nsteps):
    i = pl.program_id(0)
    b = i // nt
    t = i % nt
    V = logits_hbm.shape[-1]

    def needed(j):
        return (j % nt) * tblk < len_ref[j // nt]
    needed = lambda j: jnp.bool_(True)

    NBUF = 4
    LOOKAHEAD = NBUF - 1

    def start(j):
        return pltpu.make_async_copy(
            logits_hbm.at[j // nt, pl.ds((j % nt) * tblk, tblk)],
            vbuf.at[j % NBUF],
            sem.at[j % NBUF],
        )

    @pl.when(i == 0)
    def _():
        for j in range(min(LOOKAHEAD, nsteps)):
            @pl.when(needed(j))
            def _():
                start(j).start()

    nxt = jnp.minimum(i + LOOKAHEAD, nsteps - 1)
    @pl.when((i + LOOKAHEAD < nsteps) & needed(nxt))
    def _():
        start(nxt).start()

    @pl.when(needed(i))
    def _():
        start(i).wait()
        x = vbuf[i % NBUF]
        m = jnp.max(x, axis=-1, keepdims=True)
        iota_v = lax.broadcasted_iota(jnp.int32, (tblk, V), 1)
        am = jnp.min(jnp.where(x == m, iota_v, V), axis=-1).astype(jnp.int32)
        tidx = t * tblk + lax.broadcasted_iota(jnp.int32, (1, tblk), 1)
        out_ref[0, 0] = jnp.where(tidx < len_ref[b], am.reshape(1, tblk), BLANK)

    @pl.when(jnp.logical_not(needed(i)))
    def _():
        out_ref[0, 0] = jnp.zeros((1, tblk), jnp.int32)


def _argmax_preds(logits, logits_len, tblk=512):
    B, T, V = logits.shape
    nt = T // tblk
    nsteps = B * nt
    out = pl.pallas_call(
        functools.partial(_argmax_body, tblk=tblk, nt=nt, nsteps=nsteps),
        grid=(nsteps,),
        in_specs=[
            pl.BlockSpec(memory_space=pltpu.SMEM),
            pl.BlockSpec(memory_space=pl.ANY),
        ],
        out_specs=pl.BlockSpec((1, 1, 1, tblk), lambda i: (i // nt, i % nt, 0, 0)),
        out_shape=jax.ShapeDtypeStruct((B, nt, 1, tblk), jnp.int32),
        scratch_shapes=[
            pltpu.VMEM((4, tblk, V), jnp.float32),
            pltpu.SemaphoreType.DMA((4,)),
        ],
    )(logits_len, logits)
    return out.reshape(B, T)


def _compact_body(preds_hbm, out_hbm, len_hbm, buf, orow, cbuf, *, B, T):
    c = lax.axis_index("c")
    s = lax.axis_index("s")
    wid = s * 2 + c

    @pl.when(wid < B)
    def _():
        b = wid
        buf[pl.ds(0, NL)] = jnp.zeros((NL,), jnp.int32)
        pltpu.sync_copy(preds_hbm.at[b], buf.at[pl.ds(NL, T)])
        neg1 = jnp.full((NL,), -1, jnp.int32)

        def body(i, cnt):
            base = NL + i * NL
            v = buf[pl.ds(base, NL)]
            prev = buf[pl.ds(base - 1, NL)]
            keep = (v != prev) & (v != BLANK)
            inc = plsc.cumsum(keep.astype(jnp.int32))
            posn = cnt + inc - 1
            orow[pl.ds(i * NL, NL)] = neg1
            plsc.store_scatter(orow, [posn], v, mask=keep)
            return cnt + plsc.all_reduce_population_count(keep)

        cnt = lax.fori_loop(0, T // NL, body, jnp.zeros((NL,), jnp.int32))
        cbuf[...] = cnt
        pltpu.sync_copy(orow, out_hbm.at[b])
        pltpu.sync_copy(cbuf, len_hbm.at[b])


def _compact(preds):
    B, T = preds.shape
    mesh = plsc.VectorSubcoreMesh(
        core_axis_name="c", subcore_axis_name="s", num_cores=2, num_subcores=16
    )
    f = pl.kernel(
        functools.partial(_compact_body, B=B, T=T),
        out_type=(
            jax.ShapeDtypeStruct((B, T), jnp.int32),
            jax.ShapeDtypeStruct((B, NL), jnp.int32),
        ),
        mesh=mesh,
        scratch_types=[
            pltpu.VMEM((NL + T,), jnp.int32),
            pltpu.VMEM((T,), jnp.int32),
            pltpu.VMEM((NL,), jnp.int32),
        ],
        compiler_params=pltpu.CompilerParams(use_tc_tiling_on_sc=False, needs_layout_passes=False),
    )
    return f(preds)


def kernel(logits, logits_len):
    preds = _argmax_preds(logits, logits_len)
    out, len2d = _compact(preds)
    return out, len2d[:, 0]


# speedup vs baseline: 1.2257x; 1.0883x over previous
import functools

import jax
import jax.numpy as jnp
from jax import lax
from jax.experimental import pallas as pl
from jax.experimental.pallas import tpu as pltpu
from jax.experimental.pallas import tpu_sc as plsc

BLANK = 0
NL = 16


def _argmax_body(work_ref, w_ref, logits_hbm, preds_hbm, vbuf, obuf, sem, osem,
                 *, tblk, nt, nsteps):
    k = pl.program_id(0)
    V = logits_hbm.shape[-1]
    W = w_ref[0]

    NBUF = 4
    LOOKAHEAD = NBUF - 1

    def start_in(j):
        f = work_ref[j]
        return pltpu.make_async_copy(
            logits_hbm.at[f // nt, pl.ds((f % nt) * tblk, tblk)],
            vbuf.at[j % NBUF],
            sem.at[j % NBUF],
        )

    @pl.when(k == 0)
    def _():
        for j in range(LOOKAHEAD):
            @pl.when(j < W)
            def _():
                start_in(j).start()

    @pl.when(k + LOOKAHEAD < W)
    def _():
        start_in(k + LOOKAHEAD).start()

    @pl.when(k < W)
    def _():
        @pl.when(k >= 2)
        def _():
            pltpu.make_async_copy(obuf.at[k % 2], obuf.at[k % 2], osem.at[k % 2]).wait()

        start_in(k).wait()
        x = vbuf[k % NBUF]
        m = jnp.max(x, axis=-1, keepdims=True)
        iota_v = lax.broadcasted_iota(jnp.int32, (tblk, V), 1)
        am = jnp.min(jnp.where(x == m, iota_v, V), axis=-1).astype(jnp.int32)
        obuf[k % 2] = am.reshape(1, tblk)

        f = work_ref[k]
        pltpu.make_async_copy(
            obuf.at[k % 2],
            preds_hbm.at[pl.ds(f // nt, 1), pl.ds((f % nt) * tblk, tblk)],
            osem.at[k % 2],
        ).start()

    @pl.when(k == nsteps - 1)
    def _():
        for d in range(2):
            @pl.when(W > d)
            def _():
                j = W - 1 - d
                pltpu.make_async_copy(
                    obuf.at[j % 2], obuf.at[j % 2], osem.at[j % 2]
                ).wait()


def _argmax_preds(logits, logits_len, tblk=512):
    B, T, V = logits.shape
    nt = T // tblk
    nsteps = B * nt
    nb = jnp.clip((logits_len + tblk - 1) // tblk, 0, nt)
    flat = jnp.arange(nsteps, dtype=jnp.int32)
    active = (flat % nt) < nb[flat // nt]
    pos = jnp.where(active, jnp.cumsum(active) - 1, nsteps)
    work = jnp.zeros((nsteps + 1,), jnp.int32).at[pos].set(flat, mode="drop")[:nsteps]
    w_total = jnp.sum(active.astype(jnp.int32)).reshape(1)

    preds = pl.pallas_call(
        functools.partial(_argmax_body, tblk=tblk, nt=nt, nsteps=nsteps),
        grid=(nsteps,),
        in_specs=[
            pl.BlockSpec(memory_space=pltpu.SMEM),
            pl.BlockSpec(memory_space=pltpu.SMEM),
            pl.BlockSpec(memory_space=pl.ANY),
        ],
        out_specs=pl.BlockSpec(memory_space=pl.ANY),
        out_shape=jax.ShapeDtypeStruct((B, T), jnp.int32),
        scratch_shapes=[
            pltpu.VMEM((4, tblk, V), jnp.float32),
            pltpu.VMEM((2, 1, tblk), jnp.int32),
            pltpu.SemaphoreType.DMA((4,)),
            pltpu.SemaphoreType.DMA((2,)),
        ],
    )(work, w_total, logits)
    return preds


def _compact_body(preds_hbm, len_hbm, out_hbm, olen_hbm, buf, orow, cbuf, lbuf,
                  *, B, T):
    c = lax.axis_index("c")
    s = lax.axis_index("s")
    wid = s * 2 + c

    @pl.when(wid < B)
    def _():
        b = wid
        pltpu.sync_copy(len_hbm.at[b], lbuf)
        L = lbuf[pl.ds(0, NL)][0]
        buf[pl.ds(0, NL)] = jnp.zeros((NL,), jnp.int32)
        pltpu.sync_copy(preds_hbm.at[b], buf.at[pl.ds(NL, T)])
        neg1 = jnp.full((NL,), -1, jnp.int32)

        def fill(i, _):
            orow[pl.ds(i * NL, NL)] = neg1
            return 0

        lax.fori_loop(0, T // NL, fill, 0)

        lane = lax.iota(jnp.int32, NL)

        def body(i, cnt):
            base = NL + i * NL
            v = buf[pl.ds(base, NL)]
            prev = buf[pl.ds(base - 1, NL)]
            keep = (v != prev) & (v != BLANK) & ((i * NL + lane) < L)
            inc = plsc.cumsum(keep.astype(jnp.int32))
            posn = cnt + inc - 1
            plsc.store_scatter(orow, [posn], v, mask=keep)
            return cnt + plsc.all_reduce_population_count(keep)

        nchunk = (L + NL - 1) // NL
        cnt = lax.fori_loop(0, nchunk, body, jnp.zeros((NL,), jnp.int32))
        cbuf[...] = cnt
        pltpu.sync_copy(orow, out_hbm.at[b])
        pltpu.sync_copy(cbuf, olen_hbm.at[b])


def _compact(preds, logits_len):
    B, T = preds.shape
    len16 = jnp.broadcast_to(logits_len[:, None], (B, NL))
    mesh = plsc.VectorSubcoreMesh(
        core_axis_name="c", subcore_axis_name="s", num_cores=2, num_subcores=16
    )
    f = pl.kernel(
        functools.partial(_compact_body, B=B, T=T),
        out_type=(
            jax.ShapeDtypeStruct((B, T), jnp.int32),
            jax.ShapeDtypeStruct((B, NL), jnp.int32),
        ),
        mesh=mesh,
        scratch_types=[
            pltpu.VMEM((NL + T,), jnp.int32),
            pltpu.VMEM((T,), jnp.int32),
            pltpu.VMEM((NL,), jnp.int32),
            pltpu.VMEM((NL,), jnp.int32),
        ],
        compiler_params=pltpu.CompilerParams(use_tc_tiling_on_sc=False, needs_layout_passes=False),
    )
    return f(preds, len16)


def kernel(logits, logits_len):
    preds = _argmax_preds(logits, logits_len)
    out, len2d = _compact(preds, logits_len)
    return out, len2d[:, 0]


# speedup vs baseline: 1.2429x; 1.0141x over previous
import functools

import jax
import jax.numpy as jnp
from jax import lax
from jax.experimental import pallas as pl
from jax.experimental.pallas import tpu as pltpu
from jax.experimental.pallas import tpu_sc as plsc

BLANK = 0
NL = 16


def _argmax_body(work_ref, w_ref, logits_hbm, preds_hbm, vbuf, obuf, sem, osem,
                 *, tblk, nt, nsteps):
    k = pl.program_id(0)
    V = logits_hbm.shape[-1]
    W = w_ref[0]

    NBUF = 4
    LOOKAHEAD = NBUF - 1

    def start_in(j):
        f = work_ref[j]
        return pltpu.make_async_copy(
            logits_hbm.at[f // nt, pl.ds((f % nt) * tblk, tblk)],
            vbuf.at[j % NBUF],
            sem.at[j % NBUF],
        )

    @pl.when(k == 0)
    def _():
        for j in range(LOOKAHEAD):
            @pl.when(j < W)
            def _():
                start_in(j).start()

    @pl.when(k + LOOKAHEAD < W)
    def _():
        start_in(k + LOOKAHEAD).start()

    @pl.when(k < W)
    def _():
        @pl.when(k >= 2)
        def _():
            pltpu.make_async_copy(obuf.at[k % 2], obuf.at[k % 2], osem.at[k % 2]).wait()

        start_in(k).wait()
        x = vbuf[k % NBUF]
        m = jnp.max(x, axis=-1, keepdims=True)
        iota_v = lax.broadcasted_iota(jnp.int32, (tblk, V), 1)
        am = jnp.min(jnp.where(x == m, iota_v, V), axis=-1).astype(jnp.int32)
        obuf[k % 2] = am.reshape(1, tblk)

        f = work_ref[k]
        pltpu.make_async_copy(
            obuf.at[k % 2],
            preds_hbm.at[pl.ds(f // nt, 1), pl.ds((f % nt) * tblk, tblk)],
            osem.at[k % 2],
        ).start()

    @pl.when(k == nsteps - 1)
    def _():
        for d in range(2):
            @pl.when(W > d)
            def _():
                j = W - 1 - d
                pltpu.make_async_copy(
                    obuf.at[j % 2], obuf.at[j % 2], osem.at[j % 2]
                ).wait()


def _argmax_preds(logits, logits_len, tblk=512):
    B, T, V = logits.shape
    nt = T // tblk
    nsteps = B * nt
    nb = jnp.clip((logits_len + tblk - 1) // tblk, 0, nt)
    flat = jnp.arange(nsteps, dtype=jnp.int32)
    active = (flat % nt) < nb[flat // nt]
    pos = jnp.where(active, jnp.cumsum(active) - 1, nsteps)
    work = jnp.zeros((nsteps + 1,), jnp.int32).at[pos].set(flat, mode="drop")[:nsteps]
    w_total = jnp.sum(active.astype(jnp.int32)).reshape(1)

    preds = pl.pallas_call(
        functools.partial(_argmax_body, tblk=tblk, nt=nt, nsteps=nsteps),
        grid_spec=pltpu.PrefetchScalarGridSpec(
            num_scalar_prefetch=2,
            grid=(nsteps,),
            in_specs=[pl.BlockSpec(memory_space=pl.ANY)],
            out_specs=pl.BlockSpec(memory_space=pl.ANY),
            scratch_shapes=[
                pltpu.VMEM((4, tblk, V), jnp.float32),
                pltpu.VMEM((2, 1, tblk), jnp.int32),
                pltpu.SemaphoreType.DMA((4,)),
                pltpu.SemaphoreType.DMA((2,)),
            ],
        ),
        out_shape=jax.ShapeDtypeStruct((B, T), jnp.int32),
    )(work, w_total, logits)
    return preds


def _compact_body(preds_hbm, len_hbm, out_hbm, olen_hbm, buf, orow, cbuf, lbuf,
                  *, B, T):
    c = lax.axis_index("c")
    s = lax.axis_index("s")
    wid = s * 2 + c

    @pl.when(wid < B)
    def _():
        b = wid
        pltpu.sync_copy(len_hbm.at[b], lbuf)
        L = lbuf[pl.ds(0, NL)][0]
        buf[pl.ds(0, NL)] = jnp.zeros((NL,), jnp.int32)
        pltpu.sync_copy(preds_hbm.at[b], buf.at[pl.ds(NL, T)])
        neg1 = jnp.full((NL,), -1, jnp.int32)

        def fill(i, _):
            orow[pl.ds(i * NL, NL)] = neg1
            return 0

        lax.fori_loop(0, T // NL, fill, 0)

        lane = lax.iota(jnp.int32, NL)

        def body(i, cnt):
            base = NL + i * NL
            v = buf[pl.ds(base, NL)]
            prev = buf[pl.ds(base - 1, NL)]
            keep = (v != prev) & (v != BLANK) & ((i * NL + lane) < L)
            inc = plsc.cumsum(keep.astype(jnp.int32))
            posn = cnt + inc - 1
            plsc.store_scatter(orow, [posn], v, mask=keep)
            return cnt + plsc.all_reduce_population_count(keep)

        nchunk = (L + NL - 1) // NL
        cnt = lax.fori_loop(0, nchunk, body, jnp.zeros((NL,), jnp.int32))
        cbuf[...] = cnt
        pltpu.sync_copy(orow, out_hbm.at[b])
        pltpu.sync_copy(cbuf, olen_hbm.at[b])


def _compact(preds, logits_len):
    B, T = preds.shape
    len16 = jnp.broadcast_to(logits_len[:, None], (B, NL))
    mesh = plsc.VectorSubcoreMesh(
        core_axis_name="c", subcore_axis_name="s", num_cores=2, num_subcores=16
    )
    f = pl.kernel(
        functools.partial(_compact_body, B=B, T=T),
        out_type=(
            jax.ShapeDtypeStruct((B, T), jnp.int32),
            jax.ShapeDtypeStruct((B, NL), jnp.int32),
        ),
        mesh=mesh,
        scratch_types=[
            pltpu.VMEM((NL + T,), jnp.int32),
            pltpu.VMEM((T,), jnp.int32),
            pltpu.VMEM((NL,), jnp.int32),
            pltpu.VMEM((NL,), jnp.int32),
        ],
        compiler_params=pltpu.CompilerParams(use_tc_tiling_on_sc=False, needs_layout_passes=False),
    )
    return f(preds, len16)


def kernel(logits, logits_len):
    preds = _argmax_preds(logits, logits_len)
    out, len2d = _compact(preds, logits_len)
    return out, len2d[:, 0]


# speedup vs baseline: 1.3327x; 1.0722x over previous
import functools

import jax
import jax.numpy as jnp
from jax import lax
from jax.experimental import pallas as pl
from jax.experimental.pallas import tpu as pltpu
from jax.experimental.pallas import tpu_sc as plsc

BLANK = 0
NL = 16


def _argmax_body(work_ref, w_ref, logits_hbm, preds_hbm, vbuf, obuf, sem, osem,
                 *, tblk, nt):
    V = logits_hbm.shape[-1]
    W = w_ref[0]

    NBUF = 4
    LOOKAHEAD = NBUF - 1

    def start_in(j):
        f = work_ref[j]
        return pltpu.make_async_copy(
            logits_hbm.at[f // nt, pl.ds((f % nt) * tblk, tblk)],
            vbuf.at[j % NBUF],
            sem.at[j % NBUF],
        )

    for j in range(LOOKAHEAD):
        @pl.when(j < W)
        def _():
            start_in(j).start()

    def step(k, carry):
        @pl.when(k + LOOKAHEAD < W)
        def _():
            start_in(k + LOOKAHEAD).start()

        @pl.when(k >= 2)
        def _():
            pltpu.make_async_copy(obuf.at[k % 2], obuf.at[k % 2], osem.at[k % 2]).wait()

        start_in(k).wait()
        x = vbuf[k % NBUF]
        m = jnp.max(x, axis=-1, keepdims=True)
        iota_v = lax.broadcasted_iota(jnp.int32, (tblk, V), 1)
        am = jnp.min(jnp.where(x == m, iota_v, V), axis=-1).astype(jnp.int32)
        obuf[k % 2] = am.reshape(1, tblk)

        f = work_ref[k]
        pltpu.make_async_copy(
            obuf.at[k % 2],
            preds_hbm.at[pl.ds(f // nt, 1), pl.ds((f % nt) * tblk, tblk)],
            osem.at[k % 2],
        ).start()
        return carry

    lax.fori_loop(0, W, step, 0)

    for d in range(2):
        @pl.when(W > d)
        def _():
            j = W - 1 - d
            pltpu.make_async_copy(
                obuf.at[j % 2], obuf.at[j % 2], osem.at[j % 2]
            ).wait()


def _argmax_preds(logits, logits_len, tblk=512):
    B, T, V = logits.shape
    nt = T // tblk
    nsteps = B * nt
    nb = jnp.clip((logits_len + tblk - 1) // tblk, 0, nt)
    flat = jnp.arange(nsteps, dtype=jnp.int32)
    active = (flat % nt) < nb[flat // nt]
    pos = jnp.where(active, jnp.cumsum(active) - 1, nsteps)
    work = jnp.zeros((nsteps + 1,), jnp.int32).at[pos].set(flat, mode="drop")[:nsteps]
    w_total = jnp.sum(active.astype(jnp.int32)).reshape(1)

    preds = pl.pallas_call(
        functools.partial(_argmax_body, tblk=tblk, nt=nt),
        grid_spec=pltpu.PrefetchScalarGridSpec(
            num_scalar_prefetch=2,
            grid=(1,),
            in_specs=[pl.BlockSpec(memory_space=pl.ANY)],
            out_specs=pl.BlockSpec(memory_space=pl.ANY),
            scratch_shapes=[
                pltpu.VMEM((4, tblk, V), jnp.float32),
                pltpu.VMEM((2, 1, tblk), jnp.int32),
                pltpu.SemaphoreType.DMA((4,)),
                pltpu.SemaphoreType.DMA((2,)),
            ],
        ),
        out_shape=jax.ShapeDtypeStruct((B, T), jnp.int32),
    )(work, w_total, logits)
    return preds


def _compact_body(preds_hbm, len_hbm, out_hbm, olen_hbm, buf, orow, cbuf, lbuf,
                  *, B, T):
    c = lax.axis_index("c")
    s = lax.axis_index("s")
    wid = s * 2 + c

    @pl.when(wid < B)
    def _():
        b = wid
        pltpu.sync_copy(len_hbm.at[b], lbuf)
        L = lbuf[pl.ds(0, NL)][0]
        buf[pl.ds(0, NL)] = jnp.zeros((NL,), jnp.int32)
        pltpu.sync_copy(preds_hbm.at[b], buf.at[pl.ds(NL, T)])
        neg1 = jnp.full((NL,), -1, jnp.int32)

        def fill(i, _):
            orow[pl.ds(i * NL, NL)] = neg1
            return 0

        lax.fori_loop(0, T // NL, fill, 0)

        lane = lax.iota(jnp.int32, NL)

        def body(i, cnt):
            base = NL + i * NL
            v = buf[pl.ds(base, NL)]
            prev = buf[pl.ds(base - 1, NL)]
            keep = (v != prev) & (v != BLANK) & ((i * NL + lane) < L)
            inc = plsc.cumsum(keep.astype(jnp.int32))
            posn = cnt + inc - 1
            plsc.store_scatter(orow, [posn], v, mask=keep)
            return cnt + plsc.all_reduce_population_count(keep)

        nchunk = (L + NL - 1) // NL
        cnt = lax.fori_loop(0, nchunk, body, jnp.zeros((NL,), jnp.int32))
        cbuf[...] = cnt
        pltpu.sync_copy(orow, out_hbm.at[b])
        pltpu.sync_copy(cbuf, olen_hbm.at[b])


def _compact(preds, logits_len):
    B, T = preds.shape
    len16 = jnp.broadcast_to(logits_len[:, None], (B, NL))
    mesh = plsc.VectorSubcoreMesh(
        core_axis_name="c", subcore_axis_name="s", num_cores=2, num_subcores=16
    )
    f = pl.kernel(
        functools.partial(_compact_body, B=B, T=T),
        out_type=(
            jax.ShapeDtypeStruct((B, T), jnp.int32),
            jax.ShapeDtypeStruct((B, NL), jnp.int32),
        ),
        mesh=mesh,
        scratch_types=[
            pltpu.VMEM((NL + T,), jnp.int32),
            pltpu.VMEM((T,), jnp.int32),
            pltpu.VMEM((NL,), jnp.int32),
            pltpu.VMEM((NL,), jnp.int32),
        ],
        compiler_params=pltpu.CompilerParams(use_tc_tiling_on_sc=False, needs_layout_passes=False),
    )
    return f(preds, len16)


def kernel(logits, logits_len):
    preds = _argmax_preds(logits, logits_len)
    out, len2d = _compact(preds, logits_len)
    return out, len2d[:, 0]
